# baseline (speedup 1.0000x reference)
import functools
import math

import jax
import jax.numpy as jnp
from jax import lax
from jax.experimental import pallas as pl
from jax.experimental.pallas import tpu as pltpu
from jax.experimental.pallas import tpu_sc as plsc

D_MODEL = 64
VOCAB = 1000000
B1 = 4096
B2 = 200
NW = 32
K = 128
NCHUNK = B1 * B2 // K
PER_W = NCHUNK // NW
NBUF = 2
SCALE = math.sqrt(D_MODEL)

_mesh = plsc.VectorSubcoreMesh(core_axis_name="c", subcore_axis_name="s")


@functools.partial(
    pl.kernel,
    mesh=_mesh,
    out_type=jax.ShapeDtypeStruct((B2, D_MODEL, B1), jnp.float32),
    compiler_params=pltpu.CompilerParams(
        use_tc_tiling_on_sc=True, needs_layout_passes=False
    ),
    scratch_types=[
        pltpu.VMEM((PER_W * K,), jnp.int32),
        pltpu.VMEM((NBUF, K), jnp.int32),
        pltpu.VMEM((NBUF, K), jnp.int32),
        pltpu.VMEM((NBUF, K, 128), jnp.float32),
        pltpu.VMEM((NBUF, D_MODEL, K), jnp.float32),
        pltpu.SemaphoreType.DMA((NBUF,)),
        pltpu.SemaphoreType.DMA((NBUF,)),
    ],
)
def _embed(x_hbm, lut2_hbm, out_hbm, idx_v, idxh_v, par_v, rows_v, obuf_v,
           gsem, wsem):
    wid = lax.axis_index("s") * 2 + lax.axis_index("c")
    cid0 = wid * PER_W

    pltpu.sync_copy(x_hbm.at[wid], idx_v)

    def prep(t, b):
        for g in range(K // 16):
            sl = pl.ds(t * K + g * 16, 16)
            v = idx_v[sl]
            idxh_v[b, pl.ds(g * 16, 16)] = lax.shift_right_logical(v, 1)
            par_v[b, pl.ds(g * 16, 16)] = lax.bitwise_and(v, 1)

    def start_gather(b):
        pltpu.async_copy(lut2_hbm.at[idxh_v.at[b]], rows_v.at[b], gsem.at[b])

    def wait_gather(b):
        pltpu.make_async_copy(
            lut2_hbm.at[idxh_v.at[b]], rows_v.at[b], gsem.at[b]
        ).wait()

    def out_slice(t):
        cid = cid0 + t
        b2 = lax.shift_right_logical(cid, 5)
        bh = lax.bitwise_and(cid, 31)
        return out_hbm.at[b2, :, pl.ds(bh * K, K)]

    def start_write(t, b):
        pltpu.async_copy(obuf_v.at[b], out_slice(t), wsem.at[b])

    def wait_write(t, b):
        pltpu.make_async_copy(obuf_v.at[b], out_slice(t), wsem.at[b]).wait()

    def transpose_scale(b):
        lanes = lax.iota(jnp.int32, 16)
        rowsel = []
        colbase = []
        for g in range(K // 16):
            rowsel.append(lanes + g * 16)
            colbase.append(par_v[b, pl.ds(g * 16, 16)] * D_MODEL)

        def dbody(d, carry):
            for g in range(K // 16):
                v = plsc.load_gather(rows_v.at[b], [rowsel[g], colbase[g] + d])
                obuf_v[b, d, pl.ds(g * 16, 16)] = v * SCALE
            return carry

        lax.fori_loop(0, D_MODEL, dbody, 0)

    for b in range(NBUF):
        prep(b, b)
        start_gather(b)

    def outer(grp, carry):
        for b in range(NBUF):
            t = grp * NBUF + b
            wait_gather(b)

            @pl.when(t >= NBUF)
            def _():
                wait_write(t - NBUF, b)

            transpose_scale(b)

            @pl.when(t + NBUF < PER_W)
            def _():
                prep(t + NBUF, b)
                start_gather(b)

            start_write(t, b)
        return carry

    lax.fori_loop(0, PER_W // NBUF, outer, 0)

    for b in range(NBUF):
        wait_write(PER_W - NBUF + b, b)


@jax.jit
def kernel(x, lut):
    lut2 = lut.reshape(VOCAB // 2, 128)
    xw = x.T.reshape(NW, PER_W * K)
    out5 = _embed(xw, lut2)
    return out5.transpose(2, 0, 1)
